# Initial kernel scaffold; baseline (speedup 1.0000x reference)
import jax
import jax.numpy as jnp
from jax.experimental import pallas as pl


def kernel(outputs, workers, weight):
    raise NotImplementedError("write your pallas kernel here")


# speedup vs baseline: 1.3057x; 1.3057x over previous
import functools

import jax
import jax.numpy as jnp
from jax import lax
from jax.experimental import pallas as pl
from jax.experimental.pallas import tpu as pltpu
from jax.experimental.pallas import tpu_sc as plsc

R = 8
TILE = 128
S = TILE // R

_SC_WORKERS = 32
_SC_CHUNK = 128


def _sc_gather_rows(table, idx):
    n, d = table.shape
    m = idx.shape[0]
    per_w = m // _SC_WORKERS
    n_chunks = per_w // _SC_CHUNK
    idx3 = idx.reshape(_SC_WORKERS, n_chunks, _SC_CHUNK)
    mesh = plsc.VectorSubcoreMesh(core_axis_name="c", subcore_axis_name="s")

    @functools.partial(
        pl.kernel,
        mesh=mesh,
        out_type=jax.ShapeDtypeStruct((m, d), jnp.float32),
        scratch_types=[
            pltpu.VMEM((n_chunks, _SC_CHUNK), jnp.int32),
            pltpu.VMEM((per_w, d), jnp.float32),
            pltpu.SemaphoreType.DMA,
        ],
    )
    def gather_kernel(table_hbm, idx_hbm, out_hbm, idx_v, rows_v, sem):
        wid = lax.axis_index("s") * 2 + lax.axis_index("c")
        pltpu.sync_copy(idx_hbm.at[wid], idx_v)
        copies = [
            pltpu.async_copy(
                table_hbm.at[idx_v.at[j]],
                rows_v.at[pl.ds(j * _SC_CHUNK, _SC_CHUNK)],
                sem,
            )
            for j in range(n_chunks)
        ]
        for c in copies:
            c.wait()
        pltpu.sync_copy(rows_v, out_hbm.at[pl.ds(wid * per_w, per_w)])

    return gather_kernel(table, idx3)


def _mm_body(sw_ref, x_ref, *w_and_out):
    w_refs = w_and_out[:S]
    o_ref = w_and_out[S]
    for k in range(S):
        xblk = x_ref[pl.ds(k * R, R), :]
        o_ref[pl.ds(k * R, R), :] = lax.dot_general(
            xblk,
            w_refs[k][0],
            (((1,), (1,)), ((), ())),
            preferred_element_type=jnp.float32,
        )


def _tc_matmul_padded(slot_w, x_pad, weight):
    pad_n, l = x_pad.shape
    grid = (pad_n // TILE,)

    def w_map(i, sw, k):
        return (sw[i * S + k], 0, 0)

    grid_spec = pltpu.PrefetchScalarGridSpec(
        num_scalar_prefetch=1,
        grid=grid,
        in_specs=[pl.BlockSpec((TILE, l), lambda i, sw: (i, 0))]
        + [
            pl.BlockSpec((1, l, l), functools.partial(w_map, k=k))
            for k in range(S)
        ],
        out_specs=pl.BlockSpec((TILE, l), lambda i, sw: (i, 0)),
    )
    return pl.pallas_call(
        _mm_body,
        grid_spec=grid_spec,
        out_shape=jax.ShapeDtypeStruct((pad_n, l), jnp.float32),
    )(slot_w, x_pad, *([weight] * S))


def kernel(outputs, workers, weight):
    b, l = outputs.shape
    nw = weight.shape[0]
    workers = workers.astype(jnp.int32)

    pad_n = b + R * nw
    pad_n = ((pad_n + _SC_WORKERS * _SC_CHUNK - 1)
             // (_SC_WORKERS * _SC_CHUNK)) * (_SC_WORKERS * _SC_CHUNK)

    order = jnp.argsort(workers)
    sw = workers[order]
    counts = jnp.bincount(workers, length=nw)
    pcounts = ((counts + R - 1) // R) * R
    pstart = jnp.cumsum(pcounts) - pcounts
    start = jnp.cumsum(counts) - counts
    rank = jnp.arange(b, dtype=jnp.int32) - start[sw]
    p = (pstart[sw] + rank).astype(jnp.int32)
    src = jnp.zeros(pad_n, jnp.int32).at[p].set(order.astype(jnp.int32))
    pos = jnp.zeros(b, jnp.int32).at[order].set(p)
    slot_w = jnp.zeros(pad_n // R, jnp.int32).at[p // R].set(sw)

    x_pad = _sc_gather_rows(outputs, src)
    out_pad = _tc_matmul_padded(slot_w, x_pad, weight)
    return _sc_gather_rows(out_pad, pos)


# speedup vs baseline: 1.7741x; 1.3588x over previous
import functools

import jax
import jax.numpy as jnp
from jax import lax
from jax.experimental import pallas as pl
from jax.experimental.pallas import tpu as pltpu
from jax.experimental.pallas import tpu_sc as plsc

R = 8
TILE = 128
S = TILE // R

_SC_WORKERS = 32
_SC_CHUNK = 128


def _sc_gather_rows(table, idx):
    n, d = table.shape
    m = idx.shape[0]
    per_w = m // _SC_WORKERS
    n_chunks = per_w // _SC_CHUNK
    idx3 = idx.reshape(_SC_WORKERS, n_chunks, _SC_CHUNK)
    mesh = plsc.VectorSubcoreMesh(core_axis_name="c", subcore_axis_name="s")

    @functools.partial(
        pl.kernel,
        mesh=mesh,
        out_type=jax.ShapeDtypeStruct((m, d), jnp.float32),
        scratch_types=[
            pltpu.VMEM((n_chunks, _SC_CHUNK), jnp.int32),
            pltpu.VMEM((per_w, d), jnp.float32),
            pltpu.SemaphoreType.DMA,
        ],
    )
    def gather_kernel(table_hbm, idx_hbm, out_hbm, idx_v, rows_v, sem):
        wid = lax.axis_index("s") * 2 + lax.axis_index("c")
        pltpu.sync_copy(idx_hbm.at[wid], idx_v)
        copies = [
            pltpu.async_copy(
                table_hbm.at[idx_v.at[j]],
                rows_v.at[pl.ds(j * _SC_CHUNK, _SC_CHUNK)],
                sem,
            )
            for j in range(n_chunks)
        ]
        for c in copies:
            c.wait()
        pltpu.sync_copy(rows_v, out_hbm.at[pl.ds(wid * per_w, per_w)])

    return gather_kernel(table, idx3)


def _mm_body(sw_ref, x_ref, *w_and_out):
    w_refs = w_and_out[:S]
    o_ref = w_and_out[S]
    for k in range(S):
        xblk = x_ref[pl.ds(k * R, R), :]
        o_ref[pl.ds(k * R, R), :] = lax.dot_general(
            xblk,
            w_refs[k][0],
            (((1,), (1,)), ((), ())),
            preferred_element_type=jnp.float32,
        )


def _tc_matmul_padded(slot_w, x_pad, weight):
    pad_n, l = x_pad.shape
    grid = (pad_n // TILE,)

    def w_map(i, sw, k):
        return (sw[i * S + k], 0, 0)

    grid_spec = pltpu.PrefetchScalarGridSpec(
        num_scalar_prefetch=1,
        grid=grid,
        in_specs=[pl.BlockSpec((TILE, l), lambda i, sw: (i, 0))]
        + [
            pl.BlockSpec((1, l, l), functools.partial(w_map, k=k))
            for k in range(S)
        ],
        out_specs=pl.BlockSpec((TILE, l), lambda i, sw: (i, 0)),
    )
    return pl.pallas_call(
        _mm_body,
        grid_spec=grid_spec,
        out_shape=jax.ShapeDtypeStruct((pad_n, l), jnp.float32),
    )(slot_w, x_pad, *([weight] * S))


def kernel(outputs, workers, weight):
    b, l = outputs.shape
    nw = weight.shape[0]
    workers = workers.astype(jnp.int32)

    pad_n = b + R * nw
    pad_n = ((pad_n + _SC_WORKERS * _SC_CHUNK - 1)
             // (_SC_WORKERS * _SC_CHUNK)) * (_SC_WORKERS * _SC_CHUNK)

    order = jnp.argsort(workers)
    sw = workers[order]
    counts = jnp.bincount(workers, length=nw)
    pcounts = ((counts + R - 1) // R) * R
    pstart = jnp.cumsum(pcounts) - pcounts
    start = jnp.cumsum(counts) - counts
    rank = jnp.arange(b, dtype=jnp.int32) - start[sw]
    p = (pstart[sw] + rank).astype(jnp.int32)
    spread = jnp.arange(pad_n, dtype=jnp.int32) % b
    src = spread.at[p].set(order.astype(jnp.int32))
    pos = jnp.zeros(b, jnp.int32).at[order].set(p)
    slot_w = jnp.zeros(pad_n // R, jnp.int32).at[p // R].set(sw)

    x_pad = _sc_gather_rows(outputs, src)
    out_pad = _tc_matmul_padded(slot_w, x_pad, weight)
    return _sc_gather_rows(out_pad, pos)


# speedup vs baseline: 3.9566x; 2.2302x over previous
import functools

import jax
import jax.numpy as jnp
from jax import lax
from jax.experimental import pallas as pl
from jax.experimental.pallas import tpu as pltpu
from jax.experimental.pallas import tpu_sc as plsc

R = 8
TILE = 128
S = TILE // R

_SC_WORKERS = 32
_SC_CHUNK = 128


def _sc_permute_rows(table, gidx3, didx3, out_rows):
    d = table.shape[1]
    nwk, nch, ck = gidx3.shape
    per_w = nch * ck
    mesh = plsc.VectorSubcoreMesh(core_axis_name="c", subcore_axis_name="s")

    @functools.partial(
        pl.kernel,
        mesh=mesh,
        out_type=jax.ShapeDtypeStruct((out_rows, d), jnp.float32),
        scratch_types=[
            pltpu.VMEM((nch, ck), jnp.int32),
            pltpu.VMEM((nch, ck), jnp.int32),
            pltpu.VMEM((per_w, d), jnp.float32),
            pltpu.SemaphoreType.DMA,
            pltpu.SemaphoreType.DMA,
        ],
    )
    def permute_kernel(table_hbm, gidx_hbm, didx_hbm, out_hbm,
                       gi_v, di_v, rows_v, gsem, ssem):
        wid = lax.axis_index("s") * 2 + lax.axis_index("c")
        pltpu.sync_copy(gidx_hbm.at[wid], gi_v)
        pltpu.sync_copy(didx_hbm.at[wid], di_v)
        gathers = [
            pltpu.async_copy(
                table_hbm.at[gi_v.at[j]],
                rows_v.at[pl.ds(j * ck, ck)],
                gsem,
            )
            for j in range(nch)
        ]
        for c in gathers:
            c.wait()
        scatters = [
            pltpu.async_copy(
                rows_v.at[pl.ds(j * ck, ck)],
                out_hbm.at[di_v.at[j]],
                ssem,
            )
            for j in range(nch)
        ]
        for c in scatters:
            c.wait()

    return permute_kernel(table, gidx3, didx3)


def _mm_body(sw_ref, x_ref, *w_and_out):
    w_refs = w_and_out[:S]
    o_ref = w_and_out[S]
    for k in range(S):
        xblk = x_ref[pl.ds(k * R, R), :]
        o_ref[pl.ds(k * R, R), :] = lax.dot_general(
            xblk,
            w_refs[k][0],
            (((1,), (1,)), ((), ())),
            preferred_element_type=jnp.float32,
        )


def _tc_matmul_padded(slot_w, x_pad, weight):
    pad_n, l = x_pad.shape
    grid = (pad_n // TILE,)

    def w_map(i, sw, k):
        return (sw[i * S + k], 0, 0)

    grid_spec = pltpu.PrefetchScalarGridSpec(
        num_scalar_prefetch=1,
        grid=grid,
        in_specs=[pl.BlockSpec((TILE, l), lambda i, sw: (i, 0))]
        + [
            pl.BlockSpec((1, l, l), functools.partial(w_map, k=k))
            for k in range(S)
        ],
        out_specs=pl.BlockSpec((TILE, l), lambda i, sw: (i, 0)),
    )
    return pl.pallas_call(
        _mm_body,
        grid_spec=grid_spec,
        out_shape=jax.ShapeDtypeStruct((pad_n, l), jnp.float32),
    )(slot_w, x_pad, *([weight] * S))


def kernel(outputs, workers, weight):
    b, l = outputs.shape
    nw = weight.shape[0]
    workers = workers.astype(jnp.int32)

    pad_n = b + R * nw
    pad_n = ((pad_n + _SC_WORKERS * _SC_CHUNK - 1)
             // (_SC_WORKERS * _SC_CHUNK)) * (_SC_WORKERS * _SC_CHUNK)

    iota = jnp.arange(b, dtype=jnp.int32)
    sw, order = lax.sort_key_val(workers, iota)
    prev_w = jnp.concatenate([jnp.full((1,), -1, jnp.int32), sw[:-1]])
    is_start = sw != prev_w
    start_pos = lax.cummax(jnp.where(is_start, iota, 0))
    prev_start = jnp.concatenate([jnp.zeros((1,), jnp.int32), start_pos[:-1]])
    len_prev = iota - prev_start
    gap = jnp.where(is_start & (iota > 0), (-len_prev) % R, 0)
    p = iota + jnp.cumsum(gap, dtype=jnp.int32)
    slot_w = jnp.zeros(pad_n // R, jnp.int32).at[p // R].max(sw)

    shape3 = (_SC_WORKERS, b // (_SC_WORKERS * _SC_CHUNK), _SC_CHUNK)
    order3 = order.reshape(shape3)
    p3 = p.reshape(shape3)

    x_pad = _sc_permute_rows(outputs, order3, p3, pad_n)
    out_pad = _tc_matmul_padded(slot_w, x_pad, weight)
    return _sc_permute_rows(out_pad, p3, order3, b)


# speedup vs baseline: 5.8651x; 1.4823x over previous
import functools

import jax
import jax.numpy as jnp
from jax import lax
from jax.experimental import pallas as pl
from jax.experimental.pallas import tpu as pltpu
from jax.experimental.pallas import tpu_sc as plsc

R = 32
TILE = 512
S = TILE // R

_SC_WORKERS = 32
_SC_CHUNK = 128


def _sc_permute_rows(table, gidx3, didx3, out_rows):
    d = table.shape[1]
    nwk, nch, ck = gidx3.shape
    per_w = nch * ck
    mesh = plsc.VectorSubcoreMesh(core_axis_name="c", subcore_axis_name="s")

    @functools.partial(
        pl.kernel,
        mesh=mesh,
        out_type=jax.ShapeDtypeStruct((out_rows, d), jnp.float32),
        scratch_types=[
            pltpu.VMEM((nch, ck), jnp.int32),
            pltpu.VMEM((nch, ck), jnp.int32),
            pltpu.VMEM((per_w, d), jnp.float32),
            pltpu.SemaphoreType.DMA,
            pltpu.SemaphoreType.DMA,
        ],
    )
    def permute_kernel(table_hbm, gidx_hbm, didx_hbm, out_hbm,
                       gi_v, di_v, rows_v, gsem, ssem):
        wid = lax.axis_index("s") * 2 + lax.axis_index("c")
        pltpu.sync_copy(gidx_hbm.at[wid], gi_v)
        pltpu.sync_copy(didx_hbm.at[wid], di_v)
        gathers = [
            pltpu.async_copy(
                table_hbm.at[gi_v.at[j]],
                rows_v.at[pl.ds(j * ck, ck)],
                gsem,
            )
            for j in range(nch)
        ]
        for c in gathers:
            c.wait()
        scatters = [
            pltpu.async_copy(
                rows_v.at[pl.ds(j * ck, ck)],
                out_hbm.at[di_v.at[j]],
                ssem,
            )
            for j in range(nch)
        ]
        for c in scatters:
            c.wait()

    return permute_kernel(table, gidx3, didx3)


def _mm_body(sw_ref, x_ref, *w_and_out):
    w_refs = w_and_out[:S]
    o_ref = w_and_out[S]
    for k in range(S):
        xblk = x_ref[pl.ds(k * R, R), :]
        o_ref[pl.ds(k * R, R), :] = lax.dot_general(
            xblk,
            w_refs[k][0],
            (((1,), (1,)), ((), ())),
            preferred_element_type=jnp.float32,
        )


def _tc_matmul_padded(slot_w, x_pad, weight):
    pad_n, l = x_pad.shape
    grid = (pad_n // TILE,)

    def w_map(i, sw, k):
        return (sw[i * S + k], 0, 0)

    grid_spec = pltpu.PrefetchScalarGridSpec(
        num_scalar_prefetch=1,
        grid=grid,
        in_specs=[pl.BlockSpec((TILE, l), lambda i, sw: (i, 0))]
        + [
            pl.BlockSpec((1, l, l), functools.partial(w_map, k=k))
            for k in range(S)
        ],
        out_specs=pl.BlockSpec((TILE, l), lambda i, sw: (i, 0)),
    )
    return pl.pallas_call(
        _mm_body,
        grid_spec=grid_spec,
        out_shape=jax.ShapeDtypeStruct((pad_n, l), jnp.float32),
    )(slot_w, x_pad, *([weight] * S))


def kernel(outputs, workers, weight):
    b, l = outputs.shape
    nw = weight.shape[0]
    workers = workers.astype(jnp.int32)

    pad_n = b + R * nw
    pad_n = ((pad_n + _SC_WORKERS * _SC_CHUNK - 1)
             // (_SC_WORKERS * _SC_CHUNK)) * (_SC_WORKERS * _SC_CHUNK)

    iota = jnp.arange(b, dtype=jnp.int32)
    sw, order = lax.sort_key_val(workers, iota)
    prev_w = jnp.concatenate([jnp.full((1,), -1, jnp.int32), sw[:-1]])
    is_start = sw != prev_w
    start_pos = lax.cummax(jnp.where(is_start, iota, 0))
    prev_start = jnp.concatenate([jnp.zeros((1,), jnp.int32), start_pos[:-1]])
    len_prev = iota - prev_start
    gap = jnp.where(is_start & (iota > 0), (-len_prev) % R, 0)
    p = iota + jnp.cumsum(gap, dtype=jnp.int32)
    slot_w = jnp.zeros(pad_n // R, jnp.int32).at[p // R].max(sw)

    shape3 = (_SC_WORKERS, b // (_SC_WORKERS * _SC_CHUNK), _SC_CHUNK)
    order3 = order.reshape(shape3)
    p3 = p.reshape(shape3)

    x_pad = _sc_permute_rows(outputs, order3, p3, pad_n)
    out_pad = _tc_matmul_padded(slot_w, x_pad, weight)
    return _sc_permute_rows(out_pad, p3, order3, b)
